# baseline (speedup 1.0000x reference)
import jax
import jax.numpy as jnp
from jax import lax
from jax.experimental import pallas as pl
from jax.experimental.pallas import tpu as pltpu
from jax.experimental.pallas import tpu_sc as plsc

NC = 2
NS = 16
NW = NC * NS

KB = 8
NB = 4

RB = 1024


def _relayout_block(wt_ref, out_ref):
    dim = wt_ref.shape[0]
    eye = jnp.eye(dim, dtype=jnp.float32)
    out_ref[:, 0:dim] = jax.lax.dot_general(
        wt_ref[...], eye, (((0,), (0,)), ((), ())),
        preferred_element_type=jnp.float32)


def _relayout(wt_t):
    dim, n_rows = wt_t.shape
    return pl.pallas_call(
        _relayout_block,
        grid=((n_rows + RB - 1) // RB,),
        in_specs=[pl.BlockSpec((dim, RB), lambda j: (0, j))],
        out_specs=pl.BlockSpec((RB, 128), lambda j: (j, 0)),
        out_shape=jax.ShapeDtypeStruct((n_rows, 128), jnp.float32),
    )(wt_t)


def _make_gather(batch, fields, dim):
    rows_w = batch // NW
    n_chunks = rows_w // KB
    n_groups = n_chunks // NB
    mesh = plsc.VectorSubcoreMesh(
        core_axis_name="c", subcore_axis_name="s",
        num_cores=NC, num_subcores=NS)

    @pl.kernel(
        out_type=jax.ShapeDtypeStruct((batch, fields, dim), jnp.float32),
        mesh=mesh,
        scratch_types=[
            pltpu.VMEM((rows_w, fields), jnp.int32),
            [pltpu.VMEM((KB, fields, 128), jnp.float32) for _ in range(NB)],
            [pltpu.SemaphoreType.DMA for _ in range(NB)],
            [pltpu.SemaphoreType.DMA for _ in range(NB)],
        ],
        compiler_params=pltpu.CompilerParams(use_tc_tiling_on_sc=False),
    )
    def k(idx_hbm, table_hbm, out_hbm, idx_v, rows, gsems, ssems):
        wid = lax.axis_index("s") * NC + lax.axis_index("c")
        base_b = wid * rows_w
        pltpu.sync_copy(idx_hbm.at[pl.ds(base_b, rows_w)], idx_v)

        def group(g, carry):
            for b in range(NB):
                j = g * NB + b
                for r in range(KB):
                    pltpu.make_async_copy(
                        table_hbm.at[idx_v.at[j * KB + r]],
                        rows[b].at[r], gsems[b]).start()
            for b in range(NB):
                j = g * NB + b
                for r in range(KB):
                    pltpu.make_async_copy(
                        table_hbm.at[idx_v.at[j * KB + r]],
                        rows[b].at[r], gsems[b]).wait()
                pltpu.make_async_copy(
                    rows[b].at[:, :, pl.ds(0, dim)],
                    out_hbm.at[pl.ds(base_b + j * KB, KB)],
                    ssems[b]).start()
            for b in range(NB):
                j = g * NB + b
                pltpu.make_async_copy(
                    rows[b].at[:, :, pl.ds(0, dim)],
                    out_hbm.at[pl.ds(base_b + j * KB, KB)],
                    ssems[b]).wait()
            return carry

        lax.fori_loop(0, n_groups, group, 0)

    return k


def kernel(indices, weight):
    batch, fields = indices.shape
    dim = weight.shape[1]
    assert batch % (NW * KB * NB) == 0
    table = _relayout(weight.T)
    return _make_gather(batch, fields, dim)(indices, table)


# speedup vs baseline: 1.3516x; 1.3516x over previous
import jax
import jax.numpy as jnp
from jax import lax
from jax.experimental import pallas as pl
from jax.experimental.pallas import tpu as pltpu
from jax.experimental.pallas import tpu_sc as plsc

NC = 2
NS = 16
NW = NC * NS

KB = 8
NB = 4

RB = 8192


def _relayout_block(wt_ref, out_ref):
    dim = wt_ref.shape[0]
    eye = jnp.eye(dim, dtype=jnp.float32)
    out_ref[:, 0:dim] = jax.lax.dot_general(
        wt_ref[...], eye, (((0,), (0,)), ((), ())),
        preferred_element_type=jnp.float32,
        precision=jax.lax.Precision.HIGHEST)


def _relayout(wt_t):
    dim, n_rows = wt_t.shape
    return pl.pallas_call(
        _relayout_block,
        grid=((n_rows + RB - 1) // RB,),
        in_specs=[pl.BlockSpec((dim, RB), lambda j: (0, j))],
        out_specs=pl.BlockSpec((RB, 128), lambda j: (j, 0)),
        out_shape=jax.ShapeDtypeStruct((n_rows, 128), jnp.float32),
    )(wt_t)


def _make_gather(batch, fields, dim):
    rows_w = batch // NW
    n_chunks = rows_w // KB
    n_groups = n_chunks // NB
    mesh = plsc.VectorSubcoreMesh(
        core_axis_name="c", subcore_axis_name="s",
        num_cores=NC, num_subcores=NS)

    @pl.kernel(
        out_type=jax.ShapeDtypeStruct((batch, fields, dim), jnp.float32),
        mesh=mesh,
        scratch_types=[
            pltpu.VMEM((rows_w, fields), jnp.int32),
            [pltpu.VMEM((KB, fields, 128), jnp.float32) for _ in range(NB)],
            [pltpu.SemaphoreType.DMA for _ in range(NB)],
            [pltpu.SemaphoreType.DMA for _ in range(NB)],
        ],
        compiler_params=pltpu.CompilerParams(use_tc_tiling_on_sc=False),
    )
    def k(idx_hbm, table_hbm, out_hbm, idx_v, rows, gsems, ssems):
        wid = lax.axis_index("s") * NC + lax.axis_index("c")
        base_b = wid * rows_w
        pltpu.sync_copy(idx_hbm.at[pl.ds(base_b, rows_w)], idx_v)

        def group(g, carry):
            for b in range(NB):
                j = g * NB + b
                for r in range(KB):
                    pltpu.make_async_copy(
                        table_hbm.at[idx_v.at[j * KB + r]],
                        rows[b].at[r], gsems[b]).start()
            for b in range(NB):
                j = g * NB + b
                for r in range(KB):
                    pltpu.make_async_copy(
                        table_hbm.at[idx_v.at[j * KB + r]],
                        rows[b].at[r], gsems[b]).wait()
                pltpu.make_async_copy(
                    rows[b].at[:, :, pl.ds(0, dim)],
                    out_hbm.at[pl.ds(base_b + j * KB, KB)],
                    ssems[b]).start()
            for b in range(NB):
                j = g * NB + b
                pltpu.make_async_copy(
                    rows[b].at[:, :, pl.ds(0, dim)],
                    out_hbm.at[pl.ds(base_b + j * KB, KB)],
                    ssems[b]).wait()
            return carry

        lax.fori_loop(0, n_groups, group, 0)

    return k


def kernel(indices, weight):
    batch, fields = indices.shape
    dim = weight.shape[1]
    assert batch % (NW * KB * NB) == 0
    table = _relayout(weight.T)
    return _make_gather(batch, fields, dim)(indices, table)


# speedup vs baseline: 1.3711x; 1.0144x over previous
import jax
import jax.numpy as jnp
from jax import lax
from jax.experimental import pallas as pl
from jax.experimental.pallas import tpu as pltpu
from jax.experimental.pallas import tpu_sc as plsc

NC = 2
NS = 16
NW = NC * NS

KB = 8
NB = 4

RB = 14336


def _relayout_block(wt_ref, out_ref):
    dim = wt_ref.shape[0]
    eye = jnp.eye(dim, dtype=jnp.float32)
    out_ref[:, 0:dim] = jax.lax.dot_general(
        wt_ref[...], eye, (((0,), (0,)), ((), ())),
        preferred_element_type=jnp.float32,
        precision=jax.lax.Precision.HIGHEST)


def _relayout(wt_t):
    dim, n_rows = wt_t.shape
    return pl.pallas_call(
        _relayout_block,
        grid=((n_rows + RB - 1) // RB,),
        in_specs=[pl.BlockSpec((dim, RB), lambda j: (0, j))],
        out_specs=pl.BlockSpec((RB, 128), lambda j: (j, 0)),
        out_shape=jax.ShapeDtypeStruct((n_rows, 128), jnp.float32),
    )(wt_t)


def _make_gather(batch, fields, dim):
    rows_w = batch // NW
    n_chunks = rows_w // KB
    n_groups = n_chunks // NB
    mesh = plsc.VectorSubcoreMesh(
        core_axis_name="c", subcore_axis_name="s",
        num_cores=NC, num_subcores=NS)

    @pl.kernel(
        out_type=jax.ShapeDtypeStruct((batch, fields, dim), jnp.float32),
        mesh=mesh,
        scratch_types=[
            pltpu.VMEM((rows_w, fields), jnp.int32),
            [pltpu.VMEM((KB, fields, 128), jnp.float32) for _ in range(NB)],
            [pltpu.SemaphoreType.DMA for _ in range(NB)],
            [pltpu.SemaphoreType.DMA for _ in range(NB)],
        ],
        compiler_params=pltpu.CompilerParams(use_tc_tiling_on_sc=False),
    )
    def k(idx_hbm, table_hbm, out_hbm, idx_v, rows, gsems, ssems):
        wid = lax.axis_index("s") * NC + lax.axis_index("c")
        base_b = wid * rows_w
        pltpu.sync_copy(idx_hbm.at[pl.ds(base_b, rows_w)], idx_v)

        def group(g, carry):
            for b in range(NB):
                j = g * NB + b
                for r in range(KB):
                    pltpu.make_async_copy(
                        table_hbm.at[idx_v.at[j * KB + r]],
                        rows[b].at[r], gsems[b]).start()
            for b in range(NB):
                j = g * NB + b
                for r in range(KB):
                    pltpu.make_async_copy(
                        table_hbm.at[idx_v.at[j * KB + r]],
                        rows[b].at[r], gsems[b]).wait()
                pltpu.make_async_copy(
                    rows[b].at[:, :, pl.ds(0, dim)],
                    out_hbm.at[pl.ds(base_b + j * KB, KB)],
                    ssems[b]).start()
            for b in range(NB):
                j = g * NB + b
                pltpu.make_async_copy(
                    rows[b].at[:, :, pl.ds(0, dim)],
                    out_hbm.at[pl.ds(base_b + j * KB, KB)],
                    ssems[b]).wait()
            return carry

        lax.fori_loop(0, n_groups, group, 0)

    return k


def kernel(indices, weight):
    batch, fields = indices.shape
    dim = weight.shape[1]
    assert batch % (NW * KB * NB) == 0
    table = _relayout(weight.T)
    return _make_gather(batch, fields, dim)(indices, table)


# speedup vs baseline: 1.7158x; 1.2514x over previous
import jax
import jax.numpy as jnp
from jax import lax
from jax.experimental import pallas as pl
from jax.experimental.pallas import tpu as pltpu
from jax.experimental.pallas import tpu_sc as plsc

NC = 2
NS = 16
NW = NC * NS

KB = 8
NB = 4

RB = 14336


def _relayout_block(wt_ref, out_ref):
    dim = wt_ref.shape[0]
    out_ref[:, 0:dim] = wt_ref[...].T


def _relayout(wt_t):
    dim, n_rows = wt_t.shape
    return pl.pallas_call(
        _relayout_block,
        grid=((n_rows + RB - 1) // RB,),
        in_specs=[pl.BlockSpec((dim, RB), lambda j: (0, j))],
        out_specs=pl.BlockSpec((RB, 128), lambda j: (j, 0)),
        out_shape=jax.ShapeDtypeStruct((n_rows, 128), jnp.float32),
    )(wt_t)


def _make_gather(batch, fields, dim):
    rows_w = batch // NW
    n_chunks = rows_w // KB
    n_groups = n_chunks // NB
    mesh = plsc.VectorSubcoreMesh(
        core_axis_name="c", subcore_axis_name="s",
        num_cores=NC, num_subcores=NS)

    @pl.kernel(
        out_type=jax.ShapeDtypeStruct((batch, fields, dim), jnp.float32),
        mesh=mesh,
        scratch_types=[
            pltpu.VMEM((rows_w, fields), jnp.int32),
            [pltpu.VMEM((KB, fields, 128), jnp.float32) for _ in range(NB)],
            [pltpu.SemaphoreType.DMA for _ in range(NB)],
            [pltpu.SemaphoreType.DMA for _ in range(NB)],
        ],
        compiler_params=pltpu.CompilerParams(use_tc_tiling_on_sc=False),
    )
    def k(idx_hbm, table_hbm, out_hbm, idx_v, rows, gsems, ssems):
        wid = lax.axis_index("s") * NC + lax.axis_index("c")
        base_b = wid * rows_w
        pltpu.sync_copy(idx_hbm.at[pl.ds(base_b, rows_w)], idx_v)

        def group(g, carry):
            for b in range(NB):
                j = g * NB + b
                for r in range(KB):
                    pltpu.make_async_copy(
                        table_hbm.at[idx_v.at[j * KB + r]],
                        rows[b].at[r], gsems[b]).start()
            for b in range(NB):
                j = g * NB + b
                for r in range(KB):
                    pltpu.make_async_copy(
                        table_hbm.at[idx_v.at[j * KB + r]],
                        rows[b].at[r], gsems[b]).wait()
                pltpu.make_async_copy(
                    rows[b].at[:, :, pl.ds(0, dim)],
                    out_hbm.at[pl.ds(base_b + j * KB, KB)],
                    ssems[b]).start()
            for b in range(NB):
                j = g * NB + b
                pltpu.make_async_copy(
                    rows[b].at[:, :, pl.ds(0, dim)],
                    out_hbm.at[pl.ds(base_b + j * KB, KB)],
                    ssems[b]).wait()
            return carry

        lax.fori_loop(0, n_groups, group, 0)

    return k


def kernel(indices, weight):
    batch, fields = indices.shape
    dim = weight.shape[1]
    assert batch % (NW * KB * NB) == 0
    table = _relayout(weight.T)
    return _make_gather(batch, fields, dim)(indices, table)
